# baseline (speedup 1.0000x reference)
import functools
import math

import jax
import jax.numpy as jnp
from jax import lax
from jax.experimental import pallas as pl
from jax.experimental.pallas import tpu as pltpu
from jax.experimental.pallas import tpu_sc as plsc

N = 10000
NP = 10240
NE = 160000
NEP = 163840
D = 128
EC = 64
R_BLK = 1280
NEG = 0.2

_mesh = plsc.VectorSubcoreMesh(core_axis_name="c", subcore_axis_name="s")
_sc_params = pltpu.CompilerParams(needs_layout_passes=False)


@functools.partial(
    pl.kernel,
    mesh=_mesh,
    compiler_params=_sc_params,
    out_type=jax.ShapeDtypeStruct((NP, D), jnp.float32),
    scratch_types=[
        pltpu.VMEM((80,), jnp.int32),
        pltpu.VMEM((80, D), jnp.float32),
        pltpu.SemaphoreType.DMA,
    ],
)
def _sc_emb_gather(emb_hbm, xp_hbm, h_hbm, idxb, rowsb, sem):
    c = lax.axis_index("c")
    s = lax.axis_index("s")
    gw = c * 16 + s
    base = gw * (NP // 32)
    for k in range(NP // 32 // 80):
        b = base + k * 80
        pltpu.sync_copy(xp_hbm.at[pl.ds(b, 80)], idxb)
        pltpu.async_copy(emb_hbm.at[idxb], rowsb, sem).wait()
        pltpu.sync_copy(rowsb, h_hbm.at[pl.ds(b, 80)])


def _make_edge_kernel(dh, cout, feature_split):
    inv_sqrt = 1.0 / math.sqrt(float(cout))
    epw = NEP // 16 if feature_split else NEP // 32
    nchunks = epw // EC
    rpt = NP // 16

    scratch = [
        pltpu.VMEM((3, EC), jnp.int32),
        pltpu.VMEM((3, EC), jnp.int32),
        pltpu.VMEM((EC,), jnp.int32),
        pltpu.VMEM((EC,), jnp.int32),
        pltpu.VMEM((EC,), jnp.int32),
        pltpu.VMEM((EC,), jnp.int32),
        pltpu.VMEM((EC,), jnp.float32),
        pltpu.VMEM((EC,), jnp.float32),
        pltpu.VMEM((EC, dh), jnp.float32),
        pltpu.VMEM((EC, dh), jnp.float32),
        pltpu.VMEM((NP,), jnp.float32),
        pltpu.VMEM((NP,), jnp.float32),
        pltpu.VMEM((32,), jnp.float32),
        pltpu.VMEM((16,), jnp.float32),
        pltpu.VMEM_SHARED((NP, dh), jnp.float32),
        pltpu.SemaphoreType.DMA,
        pltpu.SemaphoreType.DMA,
        pltpu.SemaphoreType.DMA,
        pltpu.SemaphoreType.DMA,
        pltpu.SemaphoreType.DMA,
        pltpu.SemaphoreType.DMA,
    ]

    @functools.partial(
        pl.kernel,
        mesh=_mesh,
        compiler_params=_sc_params,
        out_type=[
            jax.ShapeDtypeStruct((2 * NP, dh), jnp.float32),
            jax.ShapeDtypeStruct((32, 16), jnp.float32),
        ],
        scratch_types=scratch,
    )
    def edge_kernel(x2_hbm, s2_hbm, meta_hbm,
                    out_hbm, ps_hbm,
                    mbuf0, mbuf1, sbuf0, sbuf1, dbuf0, dbuf1, wbuf0, wbuf1,
                    rows0, rows1, ssrc, sdst, cbuf,
                    psacc, accum,
                    ms0, ms1, gs0, gs1, ss0, ss1):
        mbufs = [mbuf0, mbuf1]
        sbufs = [sbuf0, sbuf1]
        dbufs = [dbuf0, dbuf1]
        wbufs = [wbuf0, wbuf1]
        rowbufs = [rows0, rows1]
        msems = [ms0, ms1]
        gsems = [gs0, gs1]
        ssems = [ss0, ss1]
        c = lax.axis_index("c")
        s = lax.axis_index("s")
        gw = c * 16 + s
        rowoff = c * NP if feature_split else 0
        rowbase = s * nchunks if feature_split else gw * nchunks

        pltpu.sync_copy(s2_hbm.at[0], ssrc)
        pltpu.sync_copy(s2_hbm.at[1], sdst)
        pltpu.sync_copy(s2_hbm.at[2, pl.ds(0, 16)], cbuf.at[pl.ds(0, 16)])
        pltpu.sync_copy(s2_hbm.at[3, pl.ds(0, 16)], cbuf.at[pl.ds(16, 16)])

        def zrow(i, _):
            for dcol in range(dh // 16):
                rows0[i, pl.ds(dcol * 16, 16)] = jnp.zeros((16,), jnp.float32)
            return 0
        lax.fori_loop(0, EC, zrow, 0)
        for k in range(rpt // EC):
            pltpu.sync_copy(rows0, accum.at[pl.ds(s * rpt + k * EC, EC)])
        psacc[...] = jnp.zeros((16,), jnp.float32)

        c1v = cbuf[pl.ds(0, 16)]
        c0v = cbuf[pl.ds(16, 16)]

        def _wait_rows(sem):
            pltpu.make_async_copy(x2_hbm.at[pl.ds(0, EC)], rows0, sem).wait()

        def _wait_meta(sem):
            pltpu.make_async_copy(meta_hbm.at[0], mbuf0, sem).wait()

        def alpha_prep(k, b):
            for g in range(EC // 16):
                sl = pl.ds(g * 16, 16)
                iv = mbufs[b][0, sl]
                dvi = mbufs[b][1, sl]
                tv = plsc.bitcast(mbufs[b][2, sl], jnp.float32)
                sv = plsc.load_gather(ssrc, [iv])
                dv = plsc.load_gather(sdst, [dvi])
                al = (sv + dv + tv * c1v + c0v) * inv_sqrt
                al = jnp.where(al >= 0, al, al * NEG)
                w = jnp.exp(-al)
                ge = (rowbase + k) * EC + g * 16 + lax.iota(jnp.int32, 16)
                w = jnp.where(ge < NE, w, 0.0)
                psacc[...] = psacc[...] + w
                wbufs[b][sl] = w
                dbufs[b][sl] = dvi
                sbufs[b][sl] = iv + rowoff
            return 0

        pltpu.sync_copy(meta_hbm.at[rowbase], mbuf0)
        alpha_prep(0, 0)
        pltpu.async_copy(x2_hbm.at[sbuf0], rows0, gsems[0])
        pltpu.async_copy(meta_hbm.at[rowbase + 1], mbuf1, msems[1])
        plsc.subcore_barrier()

        def pipe(k2, _):
            for b in range(2):
                k = k2 * 2 + b
                nb = 1 - b

                @pl.when(k >= 1)
                def _():
                    _wait_rows(ssems[nb])

                @pl.when(k + 1 < nchunks)
                def _():
                    _wait_meta(msems[nb])
                    alpha_prep(k + 1, nb)
                    pltpu.async_copy(x2_hbm.at[sbufs[nb]], rowbufs[nb],
                                     gsems[nb])

                @pl.when(k + 2 < nchunks)
                def _():
                    pltpu.async_copy(meta_hbm.at[rowbase + k + 2], mbufs[b],
                                     msems[b])

                _wait_rows(gsems[b])

                def scale(j, _):
                    for r in range(16):
                        row = j * 16 + r
                        wsplat = plsc.load_gather(
                            wbufs[b], [jnp.full((16,), row, jnp.int32)])
                        for dcol in range(dh // 16):
                            sl = pl.ds(dcol * 16, 16)
                            rowbufs[b][row, sl] = rowbufs[b][row, sl] * wsplat
                    return 0
                lax.fori_loop(0, EC // 16, scale, 0)
                pltpu.async_copy(rowbufs[b], accum.at[dbufs[b]],
                                 ssems[b], add=True)
            return 0
        lax.fori_loop(0, nchunks // 2, pipe, 0)
        _wait_rows(ssems[(nchunks - 1) % 2])
        plsc.subcore_barrier()

        pltpu.sync_copy(psacc, ps_hbm.at[gw])
        for k in range(rpt // EC):
            r0 = s * rpt + k * EC
            pltpu.sync_copy(accum.at[pl.ds(r0, EC)], rows0)
            pltpu.sync_copy(rows0, out_hbm.at[pl.ds(c * NP + r0, EC)])

    return edge_kernel


_sc_edge_l0 = _make_edge_kernel(D, 2 * D, True)
_sc_edge_l1 = _make_edge_kernel(D, D, False)


def _tc_pre0_body(h_ref, lin_ref, asrc_ref, adst_ref, tv_ref, av_ref, tb_ref,
                  x2_ref, s2_ref):
    x2f = jnp.dot(h_ref[...], lin_ref[...], preferred_element_type=jnp.float32)
    x2_ref[0] = x2f[:, :D]
    x2_ref[1] = x2f[:, D:]
    sv = jnp.sum(x2f * asrc_ref[...], axis=1)
    dv = jnp.sum(x2f * adst_ref[...], axis=1)
    c1 = jnp.sum(tv_ref[...] * av_ref[...])
    c0 = jnp.sum(tb_ref[...] * av_ref[...])
    s2_ref[...] = jnp.concatenate(
        [sv[None], dv[None],
         jnp.full((1, R_BLK), c1, jnp.float32),
         jnp.full((1, R_BLK), c0, jnp.float32),
         jnp.zeros((4, R_BLK), jnp.float32)], axis=0)


_tc_pre0 = pl.pallas_call(
    _tc_pre0_body,
    grid=(NP // R_BLK,),
    in_specs=[
        pl.BlockSpec((R_BLK, D), lambda r: (r, 0)),
        pl.BlockSpec((D, 2 * D), lambda r: (0, 0)),
        pl.BlockSpec((1, 2 * D), lambda r: (0, 0)),
        pl.BlockSpec((1, 2 * D), lambda r: (0, 0)),
        pl.BlockSpec((1, 64), lambda r: (0, 0)),
        pl.BlockSpec((1, 64), lambda r: (0, 0)),
        pl.BlockSpec((1, 64), lambda r: (0, 0)),
    ],
    out_specs=[
        pl.BlockSpec((2, R_BLK, D), lambda r: (0, r, 0)),
        pl.BlockSpec((8, R_BLK), lambda r: (0, r)),
    ],
    out_shape=[
        jax.ShapeDtypeStruct((2, NP, D), jnp.float32),
        jax.ShapeDtypeStruct((8, NP), jnp.float32),
    ],
)


def _tc_mid_body(acc_ref, ps_ref, lin_ref, asrc_ref, adst_ref, tv_ref, av_ref,
                 tb_ref, x2_ref, s2_ref):
    inv = 2.0 / jnp.sum(ps_ref[...])
    a = acc_ref[0] * inv
    a = jnp.where(a >= 0, a, a * NEG)
    b = acc_ref[1] * inv
    b = jnp.where(b >= 0, b, b * NEG)
    hfull = jnp.concatenate([a, b], axis=1)
    x2f = jnp.dot(hfull, lin_ref[...], preferred_element_type=jnp.float32)
    x2_ref[...] = x2f
    sv = jnp.sum(x2f * asrc_ref[...], axis=1)
    dv = jnp.sum(x2f * adst_ref[...], axis=1)
    c1 = jnp.sum(tv_ref[...] * av_ref[...])
    c0 = jnp.sum(tb_ref[...] * av_ref[...])
    s2_ref[...] = jnp.concatenate(
        [sv[None], dv[None],
         jnp.full((1, R_BLK), c1, jnp.float32),
         jnp.full((1, R_BLK), c0, jnp.float32),
         jnp.zeros((4, R_BLK), jnp.float32)], axis=0)


_tc_mid = pl.pallas_call(
    _tc_mid_body,
    grid=(NP // R_BLK,),
    in_specs=[
        pl.BlockSpec((2, R_BLK, D), lambda r: (0, r, 0)),
        pl.BlockSpec((32, 16), lambda r: (0, 0)),
        pl.BlockSpec((2 * D, D), lambda r: (0, 0)),
        pl.BlockSpec((1, D), lambda r: (0, 0)),
        pl.BlockSpec((1, D), lambda r: (0, 0)),
        pl.BlockSpec((1, 64), lambda r: (0, 0)),
        pl.BlockSpec((1, 64), lambda r: (0, 0)),
        pl.BlockSpec((1, 64), lambda r: (0, 0)),
    ],
    out_specs=[
        pl.BlockSpec((R_BLK, D), lambda r: (r, 0)),
        pl.BlockSpec((8, R_BLK), lambda r: (0, r)),
    ],
    out_shape=[
        jax.ShapeDtypeStruct((NP, D), jnp.float32),
        jax.ShapeDtypeStruct((8, NP), jnp.float32),
    ],
)


def _tc_final_body(acc_ref, ps_ref, hres_ref, y_ref):
    inv = 1.0 / jnp.sum(ps_ref[...])
    t = (acc_ref[0] + acc_ref[1]) * inv
    t = jnp.where(t >= 0, t, t * NEG)
    y_ref[...] = t + hres_ref[...]


_tc_final = pl.pallas_call(
    _tc_final_body,
    grid=(NP // R_BLK,),
    in_specs=[
        pl.BlockSpec((2, R_BLK, D), lambda r: (0, r, 0)),
        pl.BlockSpec((32, 16), lambda r: (0, 0)),
        pl.BlockSpec((R_BLK, D), lambda r: (r, 0)),
    ],
    out_specs=pl.BlockSpec((R_BLK, D), lambda r: (r, 0)),
    out_shape=jax.ShapeDtypeStruct((NP, D), jnp.float32),
)


def kernel(x, edge_index, edge_time, emb, te_W1, te_b1, te_W2, te_b2,
           l0_lin, l0_tW, l0_tb, l0_asrc, l0_adst, l0_atime,
           l1_lin, l1_tW, l1_tb, l1_asrc, l1_adst, l1_atime):
    xp = jnp.pad(x, (0, NP - N))
    pad_idx = N + (jnp.arange(NEP - NE, dtype=jnp.int32) % (NP - N))
    srcp = jnp.concatenate([edge_index[0], pad_idx])
    dstp = jnp.concatenate([edge_index[1], pad_idx])
    tbits = jax.lax.bitcast_convert_type(jnp.pad(edge_time, (0, NEP - NE)),
                                         jnp.int32)
    meta = jnp.stack([srcp.reshape(-1, EC), dstp.reshape(-1, EC),
                      tbits.reshape(-1, EC)], axis=1)

    h = _sc_emb_gather(emb, xp)

    x2_0, s2_0 = _tc_pre0(
        h, l0_lin,
        l0_asrc.reshape(1, 2 * D), l0_adst.reshape(1, 2 * D),
        l0_tW.reshape(1, 64), l0_atime.reshape(1, 64), l0_tb.reshape(1, 64))
    out0, ps0 = _sc_edge_l0(x2_0.reshape(2 * NP, D), s2_0, meta)

    x2_1, s2_1 = _tc_mid(
        out0.reshape(2, NP, D), ps0, l1_lin,
        l1_asrc.reshape(1, D), l1_adst.reshape(1, D),
        l1_tW.reshape(1, 64), l1_atime.reshape(1, 64), l1_tb.reshape(1, 64))
    out1, ps1 = _sc_edge_l1(x2_1, s2_1, meta)

    y = _tc_final(out1.reshape(2, NP, D), ps1, h)
    return y[:N]


# speedup vs baseline: 1.1322x; 1.1322x over previous
import functools
import math

import jax
import jax.numpy as jnp
from jax import lax
from jax.experimental import pallas as pl
from jax.experimental.pallas import tpu as pltpu
from jax.experimental.pallas import tpu_sc as plsc

N = 10000
NP = 10240
NE = 160000
NEP = 163840
D = 128
EC = 64
R_BLK = 1280
NEG = 0.2

_mesh = plsc.VectorSubcoreMesh(core_axis_name="c", subcore_axis_name="s")
_sc_params = pltpu.CompilerParams(needs_layout_passes=False)


@functools.partial(
    pl.kernel,
    mesh=_mesh,
    compiler_params=_sc_params,
    out_type=jax.ShapeDtypeStruct((NP, D), jnp.float32),
    scratch_types=[
        pltpu.VMEM((80,), jnp.int32),
        pltpu.VMEM((80,), jnp.int32),
        pltpu.VMEM((80, D), jnp.float32),
        pltpu.VMEM((80, D), jnp.float32),
        pltpu.SemaphoreType.DMA,
        pltpu.SemaphoreType.DMA,
    ],
)
def _sc_emb_gather(emb_hbm, xp_hbm, h_hbm, idxb0, idxb1, rowsb0, rowsb1,
                   sem0, sem1):
    idxbs = [idxb0, idxb1]
    rowsbs = [rowsb0, rowsb1]
    sems = [sem0, sem1]
    c = lax.axis_index("c")
    s = lax.axis_index("s")
    gw = c * 16 + s
    base = gw * (NP // 32)
    nck = NP // 32 // 80
    pltpu.sync_copy(xp_hbm.at[pl.ds(base, 80)], idxb0)
    pltpu.async_copy(emb_hbm.at[idxb0], rowsb0, sem0)
    for k in range(nck):
        b = k % 2
        nb = 1 - b
        if k + 1 < nck:
            o = base + (k + 1) * 80
            pltpu.sync_copy(xp_hbm.at[pl.ds(o, 80)], idxbs[nb])
            pltpu.async_copy(emb_hbm.at[idxbs[nb]], rowsbs[nb], sems[nb])
        pltpu.make_async_copy(emb_hbm.at[idxbs[b]], rowsbs[b], sems[b]).wait()
        pltpu.sync_copy(rowsbs[b], h_hbm.at[pl.ds(base + k * 80, 80)])


def _make_edge_kernel(dh, cout, feature_split):
    inv_sqrt = 1.0 / math.sqrt(float(cout))
    epw = NEP // 16 if feature_split else NEP // 32
    nchunks = epw // EC
    rpt = NP // 16

    scratch = [
        pltpu.VMEM((3, EC), jnp.int32),
        pltpu.VMEM((3, EC), jnp.int32),
        pltpu.VMEM((EC,), jnp.int32),
        pltpu.VMEM((EC,), jnp.int32),
        pltpu.VMEM((EC,), jnp.int32),
        pltpu.VMEM((EC,), jnp.int32),
        pltpu.VMEM((EC,), jnp.float32),
        pltpu.VMEM((EC,), jnp.float32),
        pltpu.VMEM((EC, dh), jnp.float32),
        pltpu.VMEM((EC, dh), jnp.float32),
        pltpu.VMEM((NP,), jnp.float32),
        pltpu.VMEM((NP,), jnp.float32),
        pltpu.VMEM((32,), jnp.float32),
        pltpu.VMEM((16,), jnp.float32),
        pltpu.VMEM_SHARED((NP, dh), jnp.float32),
        pltpu.SemaphoreType.DMA,
        pltpu.SemaphoreType.DMA,
        pltpu.SemaphoreType.DMA,
        pltpu.SemaphoreType.DMA,
        pltpu.SemaphoreType.DMA,
        pltpu.SemaphoreType.DMA,
    ]

    @functools.partial(
        pl.kernel,
        mesh=_mesh,
        compiler_params=_sc_params,
        out_type=[
            jax.ShapeDtypeStruct((2 * NP, dh), jnp.float32),
            jax.ShapeDtypeStruct((32, 16), jnp.float32),
        ],
        scratch_types=scratch,
    )
    def edge_kernel(x2_hbm, s2_hbm, meta_hbm,
                    out_hbm, ps_hbm,
                    mbuf0, mbuf1, sbuf0, sbuf1, dbuf0, dbuf1, wbuf0, wbuf1,
                    rows0, rows1, ssrc, sdst, cbuf,
                    psacc, accum,
                    ms0, ms1, gs0, gs1, ss0, ss1):
        mbufs = [mbuf0, mbuf1]
        sbufs = [sbuf0, sbuf1]
        dbufs = [dbuf0, dbuf1]
        wbufs = [wbuf0, wbuf1]
        rowbufs = [rows0, rows1]
        msems = [ms0, ms1]
        gsems = [gs0, gs1]
        ssems = [ss0, ss1]
        c = lax.axis_index("c")
        s = lax.axis_index("s")
        gw = c * 16 + s
        rowoff = c * NP if feature_split else 0
        rowbase = s * nchunks if feature_split else gw * nchunks

        pltpu.sync_copy(s2_hbm.at[0], ssrc)
        pltpu.sync_copy(s2_hbm.at[1], sdst)
        pltpu.sync_copy(s2_hbm.at[2, pl.ds(0, 16)], cbuf.at[pl.ds(0, 16)])
        pltpu.sync_copy(s2_hbm.at[3, pl.ds(0, 16)], cbuf.at[pl.ds(16, 16)])

        def zrow(i, _):
            for dcol in range(dh // 16):
                rows0[i, pl.ds(dcol * 16, 16)] = jnp.zeros((16,), jnp.float32)
            return 0
        lax.fori_loop(0, EC, zrow, 0)
        for k in range(rpt // EC):
            pltpu.sync_copy(rows0, accum.at[pl.ds(s * rpt + k * EC, EC)])
        psacc[...] = jnp.zeros((16,), jnp.float32)

        c1v = cbuf[pl.ds(0, 16)]
        c0v = cbuf[pl.ds(16, 16)]

        def _wait_rows(sem):
            pltpu.make_async_copy(x2_hbm.at[pl.ds(0, EC)], rows0, sem).wait()

        def _wait_meta(sem):
            pltpu.make_async_copy(meta_hbm.at[0], mbuf0, sem).wait()

        def alpha_prep(k, b):
            for g in range(EC // 16):
                sl = pl.ds(g * 16, 16)
                iv = mbufs[b][0, sl]
                dvi = mbufs[b][1, sl]
                tv = plsc.bitcast(mbufs[b][2, sl], jnp.float32)
                sv = plsc.load_gather(ssrc, [iv])
                dv = plsc.load_gather(sdst, [dvi])
                al = (sv + dv + tv * c1v + c0v) * inv_sqrt
                al = jnp.where(al >= 0, al, al * NEG)
                w = jnp.exp(-al)
                ge = (rowbase + k) * EC + g * 16 + lax.iota(jnp.int32, 16)
                w = jnp.where(ge < NE, w, 0.0)
                psacc[...] = psacc[...] + w
                wbufs[b][sl] = w
                dbufs[b][sl] = dvi
                sbufs[b][sl] = iv + rowoff
            return 0

        pltpu.sync_copy(meta_hbm.at[rowbase], mbuf0)
        alpha_prep(0, 0)
        pltpu.async_copy(x2_hbm.at[sbuf0], rows0, gsems[0])
        pltpu.async_copy(meta_hbm.at[rowbase + 1], mbuf1, msems[1])
        plsc.subcore_barrier()

        def pipe(k2, _):
            for b in range(2):
                k = k2 * 2 + b
                nb = 1 - b

                @pl.when(k >= 1)
                def _():
                    _wait_rows(ssems[nb])

                @pl.when(k + 1 < nchunks)
                def _():
                    _wait_meta(msems[nb])
                    alpha_prep(k + 1, nb)
                    pltpu.async_copy(x2_hbm.at[sbufs[nb]], rowbufs[nb],
                                     gsems[nb])

                @pl.when(k + 2 < nchunks)
                def _():
                    pltpu.async_copy(meta_hbm.at[rowbase + k + 2], mbufs[b],
                                     msems[b])

                _wait_rows(gsems[b])

                def scale(j, _):
                    wv16 = wbufs[b][pl.ds(j * 16, 16)]
                    for r in range(16):
                        wsplat = jnp.full((16,), wv16[r], jnp.float32)
                        row = j * 16 + r
                        for dcol in range(dh // 16):
                            sl = pl.ds(dcol * 16, 16)
                            rowbufs[b][row, sl] = rowbufs[b][row, sl] * wsplat
                    return 0
                lax.fori_loop(0, EC // 16, scale, 0)
                pltpu.async_copy(rowbufs[b], accum.at[dbufs[b]],
                                 ssems[b], add=True)
            return 0
        lax.fori_loop(0, nchunks // 2, pipe, 0)
        _wait_rows(ssems[(nchunks - 1) % 2])
        plsc.subcore_barrier()

        pltpu.sync_copy(psacc, ps_hbm.at[gw])
        pltpu.sync_copy(accum.at[pl.ds(s * rpt, rpt)],
                        out_hbm.at[pl.ds(c * NP + s * rpt, rpt)])

    return edge_kernel


_sc_edge_l0 = _make_edge_kernel(D, 2 * D, True)
_sc_edge_l1 = _make_edge_kernel(D, D, False)


def _tc_pre0_body(h_ref, lin_ref, asrc_ref, adst_ref, tv_ref, av_ref, tb_ref,
                  x2_ref, s2_ref):
    x2f = jnp.dot(h_ref[...], lin_ref[...], preferred_element_type=jnp.float32)
    x2_ref[0] = x2f[:, :D]
    x2_ref[1] = x2f[:, D:]
    sv = jnp.sum(x2f * asrc_ref[...], axis=1)
    dv = jnp.sum(x2f * adst_ref[...], axis=1)
    c1 = jnp.sum(tv_ref[...] * av_ref[...])
    c0 = jnp.sum(tb_ref[...] * av_ref[...])
    s2_ref[...] = jnp.concatenate(
        [sv[None], dv[None],
         jnp.full((1, R_BLK), c1, jnp.float32),
         jnp.full((1, R_BLK), c0, jnp.float32),
         jnp.zeros((4, R_BLK), jnp.float32)], axis=0)


_tc_pre0 = pl.pallas_call(
    _tc_pre0_body,
    grid=(NP // R_BLK,),
    in_specs=[
        pl.BlockSpec((R_BLK, D), lambda r: (r, 0)),
        pl.BlockSpec((D, 2 * D), lambda r: (0, 0)),
        pl.BlockSpec((1, 2 * D), lambda r: (0, 0)),
        pl.BlockSpec((1, 2 * D), lambda r: (0, 0)),
        pl.BlockSpec((1, 64), lambda r: (0, 0)),
        pl.BlockSpec((1, 64), lambda r: (0, 0)),
        pl.BlockSpec((1, 64), lambda r: (0, 0)),
    ],
    out_specs=[
        pl.BlockSpec((2, R_BLK, D), lambda r: (0, r, 0)),
        pl.BlockSpec((8, R_BLK), lambda r: (0, r)),
    ],
    out_shape=[
        jax.ShapeDtypeStruct((2, NP, D), jnp.float32),
        jax.ShapeDtypeStruct((8, NP), jnp.float32),
    ],
)


def _tc_mid_body(acc_ref, ps_ref, lin_ref, asrc_ref, adst_ref, tv_ref, av_ref,
                 tb_ref, x2_ref, s2_ref):
    inv = 2.0 / jnp.sum(ps_ref[...])
    a = acc_ref[0] * inv
    a = jnp.where(a >= 0, a, a * NEG)
    b = acc_ref[1] * inv
    b = jnp.where(b >= 0, b, b * NEG)
    hfull = jnp.concatenate([a, b], axis=1)
    x2f = jnp.dot(hfull, lin_ref[...], preferred_element_type=jnp.float32)
    x2_ref[...] = x2f
    sv = jnp.sum(x2f * asrc_ref[...], axis=1)
    dv = jnp.sum(x2f * adst_ref[...], axis=1)
    c1 = jnp.sum(tv_ref[...] * av_ref[...])
    c0 = jnp.sum(tb_ref[...] * av_ref[...])
    s2_ref[...] = jnp.concatenate(
        [sv[None], dv[None],
         jnp.full((1, R_BLK), c1, jnp.float32),
         jnp.full((1, R_BLK), c0, jnp.float32),
         jnp.zeros((4, R_BLK), jnp.float32)], axis=0)


_tc_mid = pl.pallas_call(
    _tc_mid_body,
    grid=(NP // R_BLK,),
    in_specs=[
        pl.BlockSpec((2, R_BLK, D), lambda r: (0, r, 0)),
        pl.BlockSpec((32, 16), lambda r: (0, 0)),
        pl.BlockSpec((2 * D, D), lambda r: (0, 0)),
        pl.BlockSpec((1, D), lambda r: (0, 0)),
        pl.BlockSpec((1, D), lambda r: (0, 0)),
        pl.BlockSpec((1, 64), lambda r: (0, 0)),
        pl.BlockSpec((1, 64), lambda r: (0, 0)),
        pl.BlockSpec((1, 64), lambda r: (0, 0)),
    ],
    out_specs=[
        pl.BlockSpec((R_BLK, D), lambda r: (r, 0)),
        pl.BlockSpec((8, R_BLK), lambda r: (0, r)),
    ],
    out_shape=[
        jax.ShapeDtypeStruct((NP, D), jnp.float32),
        jax.ShapeDtypeStruct((8, NP), jnp.float32),
    ],
)


def _tc_final_body(acc_ref, ps_ref, hres_ref, y_ref):
    inv = 1.0 / jnp.sum(ps_ref[...])
    t = (acc_ref[0] + acc_ref[1]) * inv
    t = jnp.where(t >= 0, t, t * NEG)
    y_ref[...] = t + hres_ref[...]


_tc_final = pl.pallas_call(
    _tc_final_body,
    grid=(NP // R_BLK,),
    in_specs=[
        pl.BlockSpec((2, R_BLK, D), lambda r: (0, r, 0)),
        pl.BlockSpec((32, 16), lambda r: (0, 0)),
        pl.BlockSpec((R_BLK, D), lambda r: (r, 0)),
    ],
    out_specs=pl.BlockSpec((R_BLK, D), lambda r: (r, 0)),
    out_shape=jax.ShapeDtypeStruct((NP, D), jnp.float32),
)


def kernel(x, edge_index, edge_time, emb, te_W1, te_b1, te_W2, te_b2,
           l0_lin, l0_tW, l0_tb, l0_asrc, l0_adst, l0_atime,
           l1_lin, l1_tW, l1_tb, l1_asrc, l1_adst, l1_atime):
    xp = jnp.pad(x, (0, NP - N))
    pad_idx = N + (jnp.arange(NEP - NE, dtype=jnp.int32) % (NP - N))
    srcp = jnp.concatenate([edge_index[0], pad_idx])
    dstp = jnp.concatenate([edge_index[1], pad_idx])
    tbits = jax.lax.bitcast_convert_type(jnp.pad(edge_time, (0, NEP - NE)),
                                         jnp.int32)
    meta = jnp.stack([srcp.reshape(-1, EC), dstp.reshape(-1, EC),
                      tbits.reshape(-1, EC)], axis=1)

    h = _sc_emb_gather(emb, xp)

    x2_0, s2_0 = _tc_pre0(
        h, l0_lin,
        l0_asrc.reshape(1, 2 * D), l0_adst.reshape(1, 2 * D),
        l0_tW.reshape(1, 64), l0_atime.reshape(1, 64), l0_tb.reshape(1, 64))
    out0, ps0 = _sc_edge_l0(x2_0.reshape(2 * NP, D), s2_0, meta)

    x2_1, s2_1 = _tc_mid(
        out0.reshape(2, NP, D), ps0, l1_lin,
        l1_asrc.reshape(1, D), l1_adst.reshape(1, D),
        l1_tW.reshape(1, 64), l1_atime.reshape(1, 64), l1_tb.reshape(1, 64))
    out1, ps1 = _sc_edge_l1(x2_1, s2_1, meta)

    y = _tc_final(out1.reshape(2, NP, D), ps1, h)
    return y[:N]


# speedup vs baseline: 1.1693x; 1.0328x over previous
import functools
import math

import jax
import jax.numpy as jnp
from jax import lax
from jax.experimental import pallas as pl
from jax.experimental.pallas import tpu as pltpu
from jax.experimental.pallas import tpu_sc as plsc

N = 10000
NP = 10240
NE = 160000
NEP = 163840
D = 128
EC = 64
R_BLK = 1280
NEG = 0.2

_mesh = plsc.VectorSubcoreMesh(core_axis_name="c", subcore_axis_name="s")
_sc_params = pltpu.CompilerParams(needs_layout_passes=False)


@functools.partial(
    pl.kernel,
    mesh=_mesh,
    compiler_params=_sc_params,
    out_type=jax.ShapeDtypeStruct((NP, D), jnp.float32),
    scratch_types=[
        pltpu.VMEM((80,), jnp.int32),
        pltpu.VMEM((80,), jnp.int32),
        pltpu.VMEM((80, D), jnp.float32),
        pltpu.VMEM((80, D), jnp.float32),
        pltpu.SemaphoreType.DMA,
        pltpu.SemaphoreType.DMA,
    ],
)
def _sc_emb_gather(emb_hbm, xp_hbm, h_hbm, idxb0, idxb1, rowsb0, rowsb1,
                   sem0, sem1):
    idxbs = [idxb0, idxb1]
    rowsbs = [rowsb0, rowsb1]
    sems = [sem0, sem1]
    c = lax.axis_index("c")
    s = lax.axis_index("s")
    gw = c * 16 + s
    base = gw * (NP // 32)
    nck = NP // 32 // 80
    pltpu.sync_copy(xp_hbm.at[pl.ds(base, 80)], idxb0)
    pltpu.async_copy(emb_hbm.at[idxb0], rowsb0, sem0)
    for k in range(nck):
        b = k % 2
        nb = 1 - b
        if k + 1 < nck:
            o = base + (k + 1) * 80
            pltpu.sync_copy(xp_hbm.at[pl.ds(o, 80)], idxbs[nb])
            pltpu.async_copy(emb_hbm.at[idxbs[nb]], rowsbs[nb], sems[nb])
        pltpu.make_async_copy(emb_hbm.at[idxbs[b]], rowsbs[b], sems[b]).wait()
        pltpu.sync_copy(rowsbs[b], h_hbm.at[pl.ds(base + k * 80, 80)])


def _make_edge_kernel(dh, cout, feature_split):
    inv_sqrt = 1.0 / math.sqrt(float(cout))
    epw = NEP // 16 if feature_split else NEP // 32
    nchunks = epw // EC
    rpt = NP // 16

    scratch = [
        pltpu.VMEM((3, EC), jnp.int32),
        pltpu.VMEM((3, EC), jnp.int32),
        pltpu.VMEM((EC,), jnp.int32),
        pltpu.VMEM((EC,), jnp.int32),
        pltpu.VMEM((EC,), jnp.int32),
        pltpu.VMEM((EC,), jnp.int32),
        pltpu.VMEM((EC,), jnp.float32),
        pltpu.VMEM((EC,), jnp.float32),
        pltpu.VMEM((EC, dh), jnp.float32),
        pltpu.VMEM((EC, dh), jnp.float32),
        pltpu.VMEM((NP,), jnp.float32),
        pltpu.VMEM((NP,), jnp.float32),
        pltpu.VMEM((32,), jnp.float32),
        pltpu.VMEM((16,), jnp.float32),
        pltpu.VMEM_SHARED((NP, dh), jnp.float32),
        pltpu.SemaphoreType.DMA,
        pltpu.SemaphoreType.DMA,
        pltpu.SemaphoreType.DMA,
        pltpu.SemaphoreType.DMA,
        pltpu.SemaphoreType.DMA,
        pltpu.SemaphoreType.DMA,
    ]

    @functools.partial(
        pl.kernel,
        mesh=_mesh,
        compiler_params=_sc_params,
        out_type=[
            jax.ShapeDtypeStruct((2 * NP, dh), jnp.float32),
            jax.ShapeDtypeStruct((32, 16), jnp.float32),
        ],
        scratch_types=scratch,
    )
    def edge_kernel(x2_hbm, s2_hbm, meta_hbm,
                    out_hbm, ps_hbm,
                    mbuf0, mbuf1, sbuf0, sbuf1, dbuf0, dbuf1, wbuf0, wbuf1,
                    rows0, rows1, ssrc, sdst, cbuf,
                    psacc, accum,
                    ms0, ms1, gs0, gs1, ss0, ss1):
        mbufs = [mbuf0, mbuf1]
        sbufs = [sbuf0, sbuf1]
        dbufs = [dbuf0, dbuf1]
        wbufs = [wbuf0, wbuf1]
        rowbufs = [rows0, rows1]
        msems = [ms0, ms1]
        gsems = [gs0, gs1]
        ssems = [ss0, ss1]
        c = lax.axis_index("c")
        s = lax.axis_index("s")
        gw = c * 16 + s
        rowoff = c * NP if feature_split else 0
        rowbase = s * nchunks if feature_split else gw * nchunks

        pltpu.sync_copy(s2_hbm.at[0], ssrc)
        pltpu.sync_copy(s2_hbm.at[1], sdst)
        pltpu.sync_copy(s2_hbm.at[2, pl.ds(0, 16)], cbuf.at[pl.ds(0, 16)])
        pltpu.sync_copy(s2_hbm.at[3, pl.ds(0, 16)], cbuf.at[pl.ds(16, 16)])

        def zrow(i, _):
            for dcol in range(dh // 16):
                rows0[i, pl.ds(dcol * 16, 16)] = jnp.zeros((16,), jnp.float32)
            return 0
        lax.fori_loop(0, EC, zrow, 0)
        for k in range(rpt // EC):
            pltpu.sync_copy(rows0, accum.at[pl.ds(s * rpt + k * EC, EC)])
        psacc[...] = jnp.zeros((16,), jnp.float32)

        c1v = cbuf[pl.ds(0, 16)]
        c0v = cbuf[pl.ds(16, 16)]

        def _wait_rows(sem):
            pltpu.make_async_copy(x2_hbm.at[pl.ds(0, EC)], rows0, sem).wait()

        def _wait_meta(sem):
            pltpu.make_async_copy(meta_hbm.at[0], mbuf0, sem).wait()

        def alpha_prep(k, b):
            for g in range(EC // 16):
                sl = pl.ds(g * 16, 16)
                iv = mbufs[b][0, sl]
                dvi = mbufs[b][1, sl]
                tv = plsc.bitcast(mbufs[b][2, sl], jnp.float32)
                sv = plsc.load_gather(ssrc, [iv])
                dv = plsc.load_gather(sdst, [dvi])
                al = (sv + dv + tv * c1v + c0v) * inv_sqrt
                al = jnp.where(al >= 0, al, al * NEG)
                w = jnp.exp(-al)
                ge = (rowbase + k) * EC + g * 16 + lax.iota(jnp.int32, 16)
                w = jnp.where(ge < NE, w, 0.0)
                psacc[...] = psacc[...] + w
                wbufs[b][sl] = w
                dbufs[b][sl] = dvi
                sbufs[b][sl] = iv + rowoff
            return 0

        pltpu.sync_copy(meta_hbm.at[rowbase], mbuf0)
        alpha_prep(0, 0)
        pltpu.async_copy(x2_hbm.at[sbuf0], rows0, gsems[0])
        pltpu.async_copy(meta_hbm.at[rowbase + 1], mbuf1, msems[1])
        plsc.subcore_barrier()

        def pipe(k2, _):
            for b in range(2):
                k = k2 * 2 + b
                nb = 1 - b

                @pl.when(k + 1 < nchunks)
                def _():
                    _wait_meta(msems[nb])
                    alpha_prep(k + 1, nb)

                @pl.when(k >= 1)
                def _():
                    _wait_rows(ssems[nb])

                @pl.when(k + 1 < nchunks)
                def _():
                    pltpu.async_copy(x2_hbm.at[sbufs[nb]], rowbufs[nb],
                                     gsems[nb])

                @pl.when(k + 2 < nchunks)
                def _():
                    pltpu.async_copy(meta_hbm.at[rowbase + k + 2], mbufs[b],
                                     msems[b])

                _wait_rows(gsems[b])

                def scale(j, _):
                    wv16 = wbufs[b][pl.ds(j * 16, 16)]
                    for r in range(16):
                        wsplat = jnp.full((16,), wv16[r], jnp.float32)
                        row = j * 16 + r
                        for dcol in range(dh // 16):
                            sl = pl.ds(dcol * 16, 16)
                            rowbufs[b][row, sl] = rowbufs[b][row, sl] * wsplat
                    return 0
                lax.fori_loop(0, EC // 16, scale, 0)
                pltpu.async_copy(rowbufs[b], accum.at[dbufs[b]],
                                 ssems[b], add=True)
            return 0
        lax.fori_loop(0, nchunks // 2, pipe, 0)
        _wait_rows(ssems[(nchunks - 1) % 2])
        plsc.subcore_barrier()

        pltpu.sync_copy(psacc, ps_hbm.at[gw])
        pltpu.sync_copy(accum.at[pl.ds(s * rpt, rpt)],
                        out_hbm.at[pl.ds(c * NP + s * rpt, rpt)])

    return edge_kernel


_sc_edge_l0 = _make_edge_kernel(D, 2 * D, True)
_sc_edge_l1 = _make_edge_kernel(D, D, False)


def _tc_pre0_body(h_ref, lin_ref, asrc_ref, adst_ref, tv_ref, av_ref, tb_ref,
                  x2_ref, s2_ref):
    x2f = jnp.dot(h_ref[...], lin_ref[...], preferred_element_type=jnp.float32)
    x2_ref[0] = x2f[:, :D]
    x2_ref[1] = x2f[:, D:]
    sv = jnp.sum(x2f * asrc_ref[...], axis=1)
    dv = jnp.sum(x2f * adst_ref[...], axis=1)
    c1 = jnp.sum(tv_ref[...] * av_ref[...])
    c0 = jnp.sum(tb_ref[...] * av_ref[...])
    s2_ref[...] = jnp.concatenate(
        [sv[None], dv[None],
         jnp.full((1, R_BLK), c1, jnp.float32),
         jnp.full((1, R_BLK), c0, jnp.float32),
         jnp.zeros((4, R_BLK), jnp.float32)], axis=0)


_tc_pre0 = pl.pallas_call(
    _tc_pre0_body,
    grid=(NP // R_BLK,),
    in_specs=[
        pl.BlockSpec((R_BLK, D), lambda r: (r, 0)),
        pl.BlockSpec((D, 2 * D), lambda r: (0, 0)),
        pl.BlockSpec((1, 2 * D), lambda r: (0, 0)),
        pl.BlockSpec((1, 2 * D), lambda r: (0, 0)),
        pl.BlockSpec((1, 64), lambda r: (0, 0)),
        pl.BlockSpec((1, 64), lambda r: (0, 0)),
        pl.BlockSpec((1, 64), lambda r: (0, 0)),
    ],
    out_specs=[
        pl.BlockSpec((2, R_BLK, D), lambda r: (0, r, 0)),
        pl.BlockSpec((8, R_BLK), lambda r: (0, r)),
    ],
    out_shape=[
        jax.ShapeDtypeStruct((2, NP, D), jnp.float32),
        jax.ShapeDtypeStruct((8, NP), jnp.float32),
    ],
)


def _tc_mid_body(acc_ref, ps_ref, lin_ref, asrc_ref, adst_ref, tv_ref, av_ref,
                 tb_ref, x2_ref, s2_ref):
    inv = 2.0 / jnp.sum(ps_ref[...])
    a = acc_ref[0] * inv
    a = jnp.where(a >= 0, a, a * NEG)
    b = acc_ref[1] * inv
    b = jnp.where(b >= 0, b, b * NEG)
    hfull = jnp.concatenate([a, b], axis=1)
    x2f = jnp.dot(hfull, lin_ref[...], preferred_element_type=jnp.float32)
    x2_ref[...] = x2f
    sv = jnp.sum(x2f * asrc_ref[...], axis=1)
    dv = jnp.sum(x2f * adst_ref[...], axis=1)
    c1 = jnp.sum(tv_ref[...] * av_ref[...])
    c0 = jnp.sum(tb_ref[...] * av_ref[...])
    s2_ref[...] = jnp.concatenate(
        [sv[None], dv[None],
         jnp.full((1, R_BLK), c1, jnp.float32),
         jnp.full((1, R_BLK), c0, jnp.float32),
         jnp.zeros((4, R_BLK), jnp.float32)], axis=0)


_tc_mid = pl.pallas_call(
    _tc_mid_body,
    grid=(NP // R_BLK,),
    in_specs=[
        pl.BlockSpec((2, R_BLK, D), lambda r: (0, r, 0)),
        pl.BlockSpec((32, 16), lambda r: (0, 0)),
        pl.BlockSpec((2 * D, D), lambda r: (0, 0)),
        pl.BlockSpec((1, D), lambda r: (0, 0)),
        pl.BlockSpec((1, D), lambda r: (0, 0)),
        pl.BlockSpec((1, 64), lambda r: (0, 0)),
        pl.BlockSpec((1, 64), lambda r: (0, 0)),
        pl.BlockSpec((1, 64), lambda r: (0, 0)),
    ],
    out_specs=[
        pl.BlockSpec((R_BLK, D), lambda r: (r, 0)),
        pl.BlockSpec((8, R_BLK), lambda r: (0, r)),
    ],
    out_shape=[
        jax.ShapeDtypeStruct((NP, D), jnp.float32),
        jax.ShapeDtypeStruct((8, NP), jnp.float32),
    ],
)


def _tc_final_body(acc_ref, ps_ref, hres_ref, y_ref):
    inv = 1.0 / jnp.sum(ps_ref[...])
    t = (acc_ref[0] + acc_ref[1]) * inv
    t = jnp.where(t >= 0, t, t * NEG)
    y_ref[...] = t + hres_ref[...]


_tc_final = pl.pallas_call(
    _tc_final_body,
    grid=(NP // R_BLK,),
    in_specs=[
        pl.BlockSpec((2, R_BLK, D), lambda r: (0, r, 0)),
        pl.BlockSpec((32, 16), lambda r: (0, 0)),
        pl.BlockSpec((R_BLK, D), lambda r: (r, 0)),
    ],
    out_specs=pl.BlockSpec((R_BLK, D), lambda r: (r, 0)),
    out_shape=jax.ShapeDtypeStruct((NP, D), jnp.float32),
)


def kernel(x, edge_index, edge_time, emb, te_W1, te_b1, te_W2, te_b2,
           l0_lin, l0_tW, l0_tb, l0_asrc, l0_adst, l0_atime,
           l1_lin, l1_tW, l1_tb, l1_asrc, l1_adst, l1_atime):
    xp = jnp.pad(x, (0, NP - N))
    pad_idx = N + (jnp.arange(NEP - NE, dtype=jnp.int32) % (NP - N))
    srcp = jnp.concatenate([edge_index[0], pad_idx])
    dstp = jnp.concatenate([edge_index[1], pad_idx])
    tbits = jax.lax.bitcast_convert_type(jnp.pad(edge_time, (0, NEP - NE)),
                                         jnp.int32)
    meta = jnp.stack([srcp.reshape(-1, EC), dstp.reshape(-1, EC),
                      tbits.reshape(-1, EC)], axis=1)

    h = _sc_emb_gather(emb, xp)

    x2_0, s2_0 = _tc_pre0(
        h, l0_lin,
        l0_asrc.reshape(1, 2 * D), l0_adst.reshape(1, 2 * D),
        l0_tW.reshape(1, 64), l0_atime.reshape(1, 64), l0_tb.reshape(1, 64))
    out0, ps0 = _sc_edge_l0(x2_0.reshape(2 * NP, D), s2_0, meta)

    x2_1, s2_1 = _tc_mid(
        out0.reshape(2, NP, D), ps0, l1_lin,
        l1_asrc.reshape(1, D), l1_adst.reshape(1, D),
        l1_tW.reshape(1, 64), l1_atime.reshape(1, 64), l1_tb.reshape(1, 64))
    out1, ps1 = _sc_edge_l1(x2_1, s2_1, meta)

    y = _tc_final(out1.reshape(2, NP, D), ps1, h)
    return y[:N]


# speedup vs baseline: 1.2377x; 1.0585x over previous
import functools
import math

import jax
import jax.numpy as jnp
from jax import lax
from jax.experimental import pallas as pl
from jax.experimental.pallas import tpu as pltpu
from jax.experimental.pallas import tpu_sc as plsc

N = 10000
NP = 10240
NE = 160000
NEP = 165888
D = 128
EC = 64
R_BLK = 1280
NEG = 0.2

_mesh = plsc.VectorSubcoreMesh(core_axis_name="c", subcore_axis_name="s")
_sc_params = pltpu.CompilerParams(needs_layout_passes=False)


@functools.partial(
    pl.kernel,
    mesh=_mesh,
    compiler_params=_sc_params,
    out_type=jax.ShapeDtypeStruct((NP, D), jnp.float32),
    scratch_types=[
        pltpu.VMEM((80,), jnp.int32),
        pltpu.VMEM((80,), jnp.int32),
        pltpu.VMEM((80, D), jnp.float32),
        pltpu.VMEM((80, D), jnp.float32),
        pltpu.SemaphoreType.DMA,
        pltpu.SemaphoreType.DMA,
    ],
)
def _sc_emb_gather(emb_hbm, xp_hbm, h_hbm, idxb0, idxb1, rowsb0, rowsb1,
                   sem0, sem1):
    idxbs = [idxb0, idxb1]
    rowsbs = [rowsb0, rowsb1]
    sems = [sem0, sem1]
    c = lax.axis_index("c")
    s = lax.axis_index("s")
    gw = c * 16 + s
    base = gw * (NP // 32)
    nck = NP // 32 // 80
    pltpu.sync_copy(xp_hbm.at[pl.ds(base, 80)], idxb0)
    pltpu.async_copy(emb_hbm.at[idxb0], rowsb0, sem0)
    for k in range(nck):
        b = k % 2
        nb = 1 - b
        if k + 1 < nck:
            o = base + (k + 1) * 80
            pltpu.sync_copy(xp_hbm.at[pl.ds(o, 80)], idxbs[nb])
            pltpu.async_copy(emb_hbm.at[idxbs[nb]], rowsbs[nb], sems[nb])
        pltpu.make_async_copy(emb_hbm.at[idxbs[b]], rowsbs[b], sems[b]).wait()
        pltpu.sync_copy(rowsbs[b], h_hbm.at[pl.ds(base + k * 80, 80)])


def _make_edge_kernel(dh, cout, feature_split):
    inv_sqrt = 1.0 / math.sqrt(float(cout))
    epw = NEP // 16 if feature_split else NEP // 32
    nchunks = epw // EC
    rpt = NP // 16

    scratch = (
        [pltpu.VMEM((3, EC), jnp.int32)] * 3 +
        [pltpu.VMEM((EC,), jnp.int32)] * 3 +
        [pltpu.VMEM((EC,), jnp.int32)] * 3 +
        [pltpu.VMEM((EC,), jnp.float32)] * 3 +
        [pltpu.VMEM((EC, dh), jnp.float32)] * 3 +
        [
            pltpu.VMEM((NP,), jnp.float32),
            pltpu.VMEM((NP,), jnp.float32),
            pltpu.VMEM((32,), jnp.float32),
            pltpu.VMEM((16,), jnp.float32),
            pltpu.VMEM_SHARED((NP, dh), jnp.float32),
        ] +
        [pltpu.SemaphoreType.DMA] * 9
    )

    @functools.partial(
        pl.kernel,
        mesh=_mesh,
        compiler_params=_sc_params,
        out_type=[
            jax.ShapeDtypeStruct((2 * NP, dh), jnp.float32),
            jax.ShapeDtypeStruct((32, 16), jnp.float32),
        ],
        scratch_types=scratch,
    )
    def edge_kernel(x2_hbm, s2_hbm, meta_hbm,
                    out_hbm, ps_hbm,
                    mbuf0, mbuf1, mbuf2, sbuf0, sbuf1, sbuf2,
                    dbuf0, dbuf1, dbuf2, wbuf0, wbuf1, wbuf2,
                    rows0, rows1, rows2, ssrc, sdst, cbuf,
                    psacc, accum,
                    ms0, ms1, ms2, gs0, gs1, gs2, ss0, ss1, ss2):
        mbufs = [mbuf0, mbuf1, mbuf2]
        sbufs = [sbuf0, sbuf1, sbuf2]
        dbufs = [dbuf0, dbuf1, dbuf2]
        wbufs = [wbuf0, wbuf1, wbuf2]
        rowbufs = [rows0, rows1, rows2]
        msems = [ms0, ms1, ms2]
        gsems = [gs0, gs1, gs2]
        ssems = [ss0, ss1, ss2]
        c = lax.axis_index("c")
        s = lax.axis_index("s")
        gw = c * 16 + s
        rowoff = c * NP if feature_split else 0
        rowbase = s * nchunks if feature_split else gw * nchunks

        pltpu.sync_copy(s2_hbm.at[0], ssrc)
        pltpu.sync_copy(s2_hbm.at[1], sdst)
        pltpu.sync_copy(s2_hbm.at[2, pl.ds(0, 16)], cbuf.at[pl.ds(0, 16)])
        pltpu.sync_copy(s2_hbm.at[3, pl.ds(0, 16)], cbuf.at[pl.ds(16, 16)])

        def zrow(i, _):
            for dcol in range(dh // 16):
                rows0[i, pl.ds(dcol * 16, 16)] = jnp.zeros((16,), jnp.float32)
            return 0
        lax.fori_loop(0, EC, zrow, 0)
        for k in range(rpt // EC):
            pltpu.sync_copy(rows0, accum.at[pl.ds(s * rpt + k * EC, EC)])
        psacc[...] = jnp.zeros((16,), jnp.float32)

        c1v = cbuf[pl.ds(0, 16)]
        c0v = cbuf[pl.ds(16, 16)]

        def _wait_rows(sem):
            pltpu.make_async_copy(x2_hbm.at[pl.ds(0, EC)], rows0, sem).wait()

        def _wait_meta(sem):
            pltpu.make_async_copy(meta_hbm.at[0], mbuf0, sem).wait()

        def alpha_prep(k, b):
            for g in range(EC // 16):
                sl = pl.ds(g * 16, 16)
                iv = mbufs[b][0, sl]
                dvi = mbufs[b][1, sl]
                tv = plsc.bitcast(mbufs[b][2, sl], jnp.float32)
                sv = plsc.load_gather(ssrc, [iv])
                dv = plsc.load_gather(sdst, [dvi])
                al = (sv + dv + tv * c1v + c0v) * inv_sqrt
                al = jnp.where(al >= 0, al, al * NEG)
                w = jnp.exp(-al)
                ge = (rowbase + k) * EC + g * 16 + lax.iota(jnp.int32, 16)
                w = jnp.where(ge < NE, w, 0.0)
                psacc[...] = psacc[...] + w
                wbufs[b][sl] = w
                dbufs[b][sl] = dvi
                sbufs[b][sl] = iv + rowoff
            return 0

        pltpu.sync_copy(meta_hbm.at[rowbase], mbuf0)
        alpha_prep(0, 0)
        pltpu.async_copy(x2_hbm.at[sbuf0], rows0, gsems[0])
        pltpu.async_copy(meta_hbm.at[rowbase + 1], mbuf1, msems[1])
        plsc.subcore_barrier()

        def pipe(k3, _):
            for b in range(3):
                k = k3 * 3 + b
                p1 = (b + 1) % 3
                p2 = (b + 2) % 3

                @pl.when(k >= 2)
                def _():
                    _wait_rows(ssems[p1])

                @pl.when(k + 1 < nchunks)
                def _():
                    _wait_meta(msems[p1])
                    alpha_prep(k + 1, p1)
                    pltpu.async_copy(x2_hbm.at[sbufs[p1]], rowbufs[p1],
                                     gsems[p1])

                @pl.when(k + 2 < nchunks)
                def _():
                    pltpu.async_copy(meta_hbm.at[rowbase + k + 2], mbufs[p2],
                                     msems[p2])

                _wait_rows(gsems[b])

                def scale(j, _):
                    wv16 = wbufs[b][pl.ds(j * 16, 16)]
                    for r in range(16):
                        wsplat = jnp.full((16,), wv16[r], jnp.float32)
                        row = j * 16 + r
                        for dcol in range(dh // 16):
                            sl = pl.ds(dcol * 16, 16)
                            rowbufs[b][row, sl] = rowbufs[b][row, sl] * wsplat
                    return 0
                lax.fori_loop(0, EC // 16, scale, 0)
                pltpu.async_copy(rowbufs[b], accum.at[dbufs[b]],
                                 ssems[b], add=True)
            return 0
        lax.fori_loop(0, nchunks // 3, pipe, 0)
        _wait_rows(ssems[(nchunks - 2) % 3])
        _wait_rows(ssems[(nchunks - 1) % 3])
        plsc.subcore_barrier()

        pltpu.sync_copy(psacc, ps_hbm.at[gw])
        pltpu.sync_copy(accum.at[pl.ds(s * rpt, rpt)],
                        out_hbm.at[pl.ds(c * NP + s * rpt, rpt)])

    return edge_kernel


_sc_edge_l0 = _make_edge_kernel(D, 2 * D, True)
_sc_edge_l1 = _make_edge_kernel(D, D, False)


def _tc_pre0_body(h_ref, lin_ref, asrc_ref, adst_ref, tv_ref, av_ref, tb_ref,
                  x2_ref, s2_ref):
    x2f = jnp.dot(h_ref[...], lin_ref[...], preferred_element_type=jnp.float32)
    x2_ref[0] = x2f[:, :D]
    x2_ref[1] = x2f[:, D:]
    sv = jnp.sum(x2f * asrc_ref[...], axis=1)
    dv = jnp.sum(x2f * adst_ref[...], axis=1)
    c1 = jnp.sum(tv_ref[...] * av_ref[...])
    c0 = jnp.sum(tb_ref[...] * av_ref[...])
    s2_ref[...] = jnp.concatenate(
        [sv[None], dv[None],
         jnp.full((1, R_BLK), c1, jnp.float32),
         jnp.full((1, R_BLK), c0, jnp.float32),
         jnp.zeros((4, R_BLK), jnp.float32)], axis=0)


_tc_pre0 = pl.pallas_call(
    _tc_pre0_body,
    grid=(NP // R_BLK,),
    in_specs=[
        pl.BlockSpec((R_BLK, D), lambda r: (r, 0)),
        pl.BlockSpec((D, 2 * D), lambda r: (0, 0)),
        pl.BlockSpec((1, 2 * D), lambda r: (0, 0)),
        pl.BlockSpec((1, 2 * D), lambda r: (0, 0)),
        pl.BlockSpec((1, 64), lambda r: (0, 0)),
        pl.BlockSpec((1, 64), lambda r: (0, 0)),
        pl.BlockSpec((1, 64), lambda r: (0, 0)),
    ],
    out_specs=[
        pl.BlockSpec((2, R_BLK, D), lambda r: (0, r, 0)),
        pl.BlockSpec((8, R_BLK), lambda r: (0, r)),
    ],
    out_shape=[
        jax.ShapeDtypeStruct((2, NP, D), jnp.float32),
        jax.ShapeDtypeStruct((8, NP), jnp.float32),
    ],
)


def _tc_mid_body(acc_ref, ps_ref, lin_ref, asrc_ref, adst_ref, tv_ref, av_ref,
                 tb_ref, x2_ref, s2_ref):
    inv = 2.0 / jnp.sum(ps_ref[...])
    a = acc_ref[0] * inv
    a = jnp.where(a >= 0, a, a * NEG)
    b = acc_ref[1] * inv
    b = jnp.where(b >= 0, b, b * NEG)
    hfull = jnp.concatenate([a, b], axis=1)
    x2f = jnp.dot(hfull, lin_ref[...], preferred_element_type=jnp.float32)
    x2_ref[...] = x2f
    sv = jnp.sum(x2f * asrc_ref[...], axis=1)
    dv = jnp.sum(x2f * adst_ref[...], axis=1)
    c1 = jnp.sum(tv_ref[...] * av_ref[...])
    c0 = jnp.sum(tb_ref[...] * av_ref[...])
    s2_ref[...] = jnp.concatenate(
        [sv[None], dv[None],
         jnp.full((1, R_BLK), c1, jnp.float32),
         jnp.full((1, R_BLK), c0, jnp.float32),
         jnp.zeros((4, R_BLK), jnp.float32)], axis=0)


_tc_mid = pl.pallas_call(
    _tc_mid_body,
    grid=(NP // R_BLK,),
    in_specs=[
        pl.BlockSpec((2, R_BLK, D), lambda r: (0, r, 0)),
        pl.BlockSpec((32, 16), lambda r: (0, 0)),
        pl.BlockSpec((2 * D, D), lambda r: (0, 0)),
        pl.BlockSpec((1, D), lambda r: (0, 0)),
        pl.BlockSpec((1, D), lambda r: (0, 0)),
        pl.BlockSpec((1, 64), lambda r: (0, 0)),
        pl.BlockSpec((1, 64), lambda r: (0, 0)),
        pl.BlockSpec((1, 64), lambda r: (0, 0)),
    ],
    out_specs=[
        pl.BlockSpec((R_BLK, D), lambda r: (r, 0)),
        pl.BlockSpec((8, R_BLK), lambda r: (0, r)),
    ],
    out_shape=[
        jax.ShapeDtypeStruct((NP, D), jnp.float32),
        jax.ShapeDtypeStruct((8, NP), jnp.float32),
    ],
)


def _tc_final_body(acc_ref, ps_ref, hres_ref, y_ref):
    inv = 1.0 / jnp.sum(ps_ref[...])
    t = (acc_ref[0] + acc_ref[1]) * inv
    t = jnp.where(t >= 0, t, t * NEG)
    y_ref[...] = t + hres_ref[...]


_tc_final = pl.pallas_call(
    _tc_final_body,
    grid=(NP // R_BLK,),
    in_specs=[
        pl.BlockSpec((2, R_BLK, D), lambda r: (0, r, 0)),
        pl.BlockSpec((32, 16), lambda r: (0, 0)),
        pl.BlockSpec((R_BLK, D), lambda r: (r, 0)),
    ],
    out_specs=pl.BlockSpec((R_BLK, D), lambda r: (r, 0)),
    out_shape=jax.ShapeDtypeStruct((NP, D), jnp.float32),
)


def kernel(x, edge_index, edge_time, emb, te_W1, te_b1, te_W2, te_b2,
           l0_lin, l0_tW, l0_tb, l0_asrc, l0_adst, l0_atime,
           l1_lin, l1_tW, l1_tb, l1_asrc, l1_adst, l1_atime):
    xp = jnp.pad(x, (0, NP - N))
    pad_idx = N + (jnp.arange(NEP - NE, dtype=jnp.int32) % (NP - N))
    srcp = jnp.concatenate([edge_index[0], pad_idx])
    dstp = jnp.concatenate([edge_index[1], pad_idx])
    tbits = jax.lax.bitcast_convert_type(jnp.pad(edge_time, (0, NEP - NE)),
                                         jnp.int32)
    meta = jnp.stack([srcp.reshape(-1, EC), dstp.reshape(-1, EC),
                      tbits.reshape(-1, EC)], axis=1)

    h = _sc_emb_gather(emb, xp)

    x2_0, s2_0 = _tc_pre0(
        h, l0_lin,
        l0_asrc.reshape(1, 2 * D), l0_adst.reshape(1, 2 * D),
        l0_tW.reshape(1, 64), l0_atime.reshape(1, 64), l0_tb.reshape(1, 64))
    out0, ps0 = _sc_edge_l0(x2_0.reshape(2 * NP, D), s2_0, meta)

    x2_1, s2_1 = _tc_mid(
        out0.reshape(2, NP, D), ps0, l1_lin,
        l1_asrc.reshape(1, D), l1_adst.reshape(1, D),
        l1_tW.reshape(1, 64), l1_atime.reshape(1, 64), l1_tb.reshape(1, 64))
    out1, ps1 = _sc_edge_l1(x2_1, s2_1, meta)

    y = _tc_final(out1.reshape(2, NP, D), ps1, h)
    return y[:N]
